# baseline (speedup 1.0000x reference)
import functools

import jax
import jax.numpy as jnp
from jax.experimental import pallas as pl
from jax.experimental.pallas import tpu as pltpu


def _gat_body(x_ref, adj_ref, pos_ref, w_ref, a_ref, wpt_ref, bpos_ref,
              o_ref, wh_scr, BR):
    i = pl.program_id(1)
    N = x_ref.shape[1]

    @pl.when(i == 0)
    def _():
        wh_scr[...] = jnp.dot(x_ref[0], w_ref[...],
                              preferred_element_type=jnp.float32)

    wh = wh_scr[...]
    rows = wh_scr[pl.ds(i * BR, BR), :]

    f1 = jnp.dot(rows, a_ref[:, 0:1],
                 preferred_element_type=jnp.float32)
    f2 = jnp.dot(wh, a_ref[:, 1:2],
                 preferred_element_type=jnp.float32)
    e = f1 + f2.reshape(1, N)
    e = jnp.where(e >= 0, e, 0.2 * e)

    adj = adj_ref[0]
    e = jnp.where(adj > 0, e, -9.0e15)
    m = jnp.max(e, axis=1, keepdims=True)
    p = jnp.exp(e - m)
    att = p / jnp.sum(p, axis=1, keepdims=True)

    h = jnp.dot(att, wh, preferred_element_type=jnp.float32)

    pe = jnp.dot(pos_ref[0], wpt_ref[...],
                 preferred_element_type=jnp.float32) + bpos_ref[...]
    pe = jnp.maximum(pe, 0.0)

    h = h + pe
    o_ref[0] = jnp.where(h > 0, h, jnp.exp(jnp.minimum(h, 0.0)) - 1.0)


@jax.jit
def kernel(x, pos, adj, W, a, W_pos, b_pos):
    B, N, F_in = x.shape
    F_out = W.shape[1]
    BR = 256

    a_pair = jnp.concatenate([a[:F_out], a[F_out:]], axis=1)
    wpt = W_pos.T
    bpos = b_pos.reshape(1, F_out)

    grid = (B, N // BR)
    out = pl.pallas_call(
        functools.partial(_gat_body, BR=BR),
        grid=grid,
        in_specs=[
            pl.BlockSpec((1, N, F_in), lambda b, i: (b, 0, 0)),
            pl.BlockSpec((1, BR, N), lambda b, i: (b, i, 0)),
            pl.BlockSpec((1, BR, 3), lambda b, i: (b, i, 0)),
            pl.BlockSpec((F_in, F_out), lambda b, i: (0, 0)),
            pl.BlockSpec((F_out, 2), lambda b, i: (0, 0)),
            pl.BlockSpec((3, F_out), lambda b, i: (0, 0)),
            pl.BlockSpec((1, F_out), lambda b, i: (0, 0)),
        ],
        out_specs=pl.BlockSpec((1, BR, F_out), lambda b, i: (b, i, 0)),
        out_shape=jax.ShapeDtypeStruct((B, N, F_out), jnp.float32),
        scratch_shapes=[pltpu.VMEM((N, F_out), jnp.float32)],
        compiler_params=pltpu.CompilerParams(
            dimension_semantics=("parallel", "arbitrary"),
        ),
    )(x, adj, pos, W, a_pair, wpt, bpos)
    return out


# speedup vs baseline: 1.0147x; 1.0147x over previous
import functools

import jax
import jax.numpy as jnp
from jax.experimental import pallas as pl
from jax.experimental.pallas import tpu as pltpu


def _gat_body(x_ref, adj_ref, pos_ref, w_ref, a_ref, wpt_ref, bpos_ref,
              o_ref, wh_scr, BR):
    i = pl.program_id(1)
    N = x_ref.shape[1]

    @pl.when(i == 0)
    def _():
        wh_scr[...] = jnp.dot(x_ref[0], w_ref[...],
                              preferred_element_type=jnp.float32)

    wh = wh_scr[...]
    rows = wh_scr[pl.ds(i * BR, BR), :]

    f1 = jnp.dot(rows, a_ref[:, 0:1],
                 preferred_element_type=jnp.float32)
    f2 = jnp.dot(wh, a_ref[:, 1:2],
                 preferred_element_type=jnp.float32)
    e = f1 + f2.reshape(1, N)
    e = jnp.where(e >= 0, e, 0.2 * e)

    adj = adj_ref[0]
    e = jnp.where(adj > 0, e, -9.0e15)
    m = jnp.max(e, axis=1, keepdims=True)
    p = jnp.exp(e - m)
    s = jnp.sum(p, axis=1, keepdims=True)

    h = jnp.dot(p, wh, preferred_element_type=jnp.float32) / s

    pe = jnp.dot(pos_ref[0], wpt_ref[...],
                 preferred_element_type=jnp.float32) + bpos_ref[...]
    pe = jnp.maximum(pe, 0.0)

    h = h + pe
    o_ref[0] = jnp.where(h > 0, h, jnp.exp(jnp.minimum(h, 0.0)) - 1.0)


@jax.jit
def kernel(x, pos, adj, W, a, W_pos, b_pos):
    B, N, F_in = x.shape
    F_out = W.shape[1]
    BR = 256

    a_pair = jnp.concatenate([a[:F_out], a[F_out:]], axis=1)
    wpt = W_pos.T
    bpos = b_pos.reshape(1, F_out)

    grid = (B, N // BR)
    out = pl.pallas_call(
        functools.partial(_gat_body, BR=BR),
        grid=grid,
        in_specs=[
            pl.BlockSpec((1, N, F_in), lambda b, i: (b, 0, 0)),
            pl.BlockSpec((1, BR, N), lambda b, i: (b, i, 0)),
            pl.BlockSpec((1, BR, 3), lambda b, i: (b, i, 0)),
            pl.BlockSpec((F_in, F_out), lambda b, i: (0, 0)),
            pl.BlockSpec((F_out, 2), lambda b, i: (0, 0)),
            pl.BlockSpec((3, F_out), lambda b, i: (0, 0)),
            pl.BlockSpec((1, F_out), lambda b, i: (0, 0)),
        ],
        out_specs=pl.BlockSpec((1, BR, F_out), lambda b, i: (b, i, 0)),
        out_shape=jax.ShapeDtypeStruct((B, N, F_out), jnp.float32),
        scratch_shapes=[pltpu.VMEM((N, F_out), jnp.float32)],
        compiler_params=pltpu.CompilerParams(
            dimension_semantics=("parallel", "arbitrary"),
        ),
    )(x, adj, pos, W, a_pair, wpt, bpos)
    return out
